# baseline (speedup 1.0000x reference)
import functools

import jax
import jax.numpy as jnp
from jax import lax
from jax.experimental import pallas as pl
from jax.experimental.pallas import tpu as pltpu
from jax.experimental.pallas import tpu_sc as plsc

E = 1_600_000
K = 20
OUT_D = 42
NW = 32
EPW = E // NW
CHUNK = 400
NCHUNK = EPW // CHUNK
GROUPS = CHUNK // 16

_SMAGIC = float(1.5 * 2**23)
_INV2PI = 0.15915494309189535
_TPI_HI = 6.28125
_TPI_LO = 0.0028353071795864769
_C0 = 9.99999995e-01
_C1 = -1.66666646e-01
_C2 = 8.33331039e-03
_C3 = -1.98401553e-04
_C4 = 2.75294535e-06
_C5 = -2.46769610e-08
_C6 = 1.34514372e-10


def _sin(x):
    y = x * _INV2PI
    n = (y + _SMAGIC) - _SMAGIC
    r = x - n * _TPI_HI
    r = r - n * _TPI_LO
    r2 = r * r
    p = _C6
    p = p * r2 + _C5
    p = p * r2 + _C4
    p = p * r2 + _C3
    p = p * r2 + _C2
    p = p * r2 + _C1
    p = p * r2 + _C0
    return r * p


_MESH = plsc.VectorSubcoreMesh(core_axis_name="c", subcore_axis_name="s")


@functools.partial(
    pl.kernel,
    mesh=_MESH,
    compiler_params=pltpu.CompilerParams(needs_layout_passes=False),
    out_type=jax.ShapeDtypeStruct((2, NW, 16), jnp.float32),
    scratch_types=[
        pltpu.VMEM((CHUNK, 3), jnp.float32),
        pltpu.VMEM((16,), jnp.float32),
        pltpu.VMEM((16,), jnp.float32),
    ],
)
def _reduce_kernel(attrs_hbm, part_hbm, buf, mscr, sscr):
    wid = lax.axis_index("s") * 2 + lax.axis_index("c")
    base = wid * EPW
    iota = lax.iota(jnp.int32, 16)
    col1 = iota * 0 + 1

    def chunk_body(c, carry):
        m, s = carry
        pltpu.sync_copy(attrs_hbm.at[pl.ds(base + c * CHUNK, CHUNK)], buf)

        def gmax(g, cm):
            v = plsc.load_gather(buf, [iota + g * 16, col1])
            return jnp.maximum(cm, v)

        cm = lax.fori_loop(0, GROUPS, gmax, jnp.full((16,), -1e30, jnp.float32))
        mnew = jnp.maximum(m, cm)
        s = s * jnp.exp(m - mnew)

        def gsum(g, acc):
            v = plsc.load_gather(buf, [iota + g * 16, col1])
            return acc + jnp.exp(v - mnew)

        s = lax.fori_loop(0, GROUPS, gsum, s)
        return mnew, s

    m0 = jnp.full((16,), -1e30, jnp.float32)
    s0 = jnp.zeros((16,), jnp.float32)
    m, s = lax.fori_loop(0, NCHUNK, chunk_body, (m0, s0))
    mscr[...] = m
    sscr[...] = s
    pltpu.sync_copy(mscr, part_hbm.at[0, wid])
    pltpu.sync_copy(sscr, part_hbm.at[1, wid])


@functools.partial(
    pl.kernel,
    mesh=_MESH,
    compiler_params=pltpu.CompilerParams(needs_layout_passes=False),
    out_type=jax.ShapeDtypeStruct((E, OUT_D), jnp.float32),
    scratch_types=[
        pltpu.VMEM((CHUNK, 3), jnp.float32),
        pltpu.VMEM((CHUNK, OUT_D), jnp.float32),
        pltpu.VMEM((2, NW, 16), jnp.float32),
        pltpu.VMEM((96, 16), jnp.float32),
    ],
)
def _main_kernel(attrs_hbm, consts_hbm, part_hbm, out_hbm, inb, outb, pv, cv):
    wid = lax.axis_index("s") * 2 + lax.axis_index("c")
    base = wid * EPW
    pltpu.sync_copy(part_hbm, pv)
    pltpu.sync_copy(consts_hbm, cv)

    m = pv[0, 0]
    s = pv[1, 0]
    for i in range(1, NW):
        mi = pv[0, i]
        si = pv[1, i]
        mn = jnp.maximum(m, mi)
        s = s * jnp.exp(m - mn) + si * jnp.exp(mi - mn)
        m = mn
    m_g = jnp.broadcast_to(jnp.max(m), (16,))
    s_g = jnp.broadcast_to(jnp.sum(s * jnp.exp(m - m_g)), (16,))
    inv_s = 1.0 / s_g

    iota = lax.iota(jnp.int32, 16)
    zero = iota * 0
    cols = [zero + j for j in range(OUT_D)]
    w0 = cv[0]
    b0 = cv[1]
    wv = [cv[2 + k] for k in range(K)]
    bv = [cv[22 + k] for k in range(K)]
    e0v = [cv[42 + j] for j in range(K)]
    dev = [cv[62 + j] for j in range(K)]

    def chunk_body(c, carry):
        cbase = base + c * CHUNK
        pltpu.sync_copy(attrs_hbm.at[pl.ds(cbase, CHUNK)], inb)

        def group(g, carry2):
            rows = iota + g * 16
            ty = plsc.load_gather(inb, [rows, cols[0]])
            fr = plsc.load_gather(inb, [rows, cols[1]])
            t = plsc.load_gather(inb, [rows, cols[2]])
            plsc.store_scatter(outb, [rows, cols[0]], t * w0 + b0)
            for k in range(K):
                sv = _sin(t * wv[k] + bv[k])
                plsc.store_scatter(outb, [rows, cols[1 + k]], sv)
            e = jnp.exp(fr - m_g) * inv_s
            wcol = jnp.where(ty == 1.0, 1.0, e)
            plsc.store_scatter(outb, [rows, cols[21]], wcol)
            for j in range(K):
                col = ty * dev[j] + e0v[j]
                plsc.store_scatter(outb, [rows, cols[22 + j]], col)
            return carry2

        lax.fori_loop(0, GROUPS, group, 0)
        pltpu.sync_copy(outb, out_hbm.at[pl.ds(cbase, CHUNK)])
        return carry

    lax.fori_loop(0, NCHUNK, chunk_body, 0)


TCB = 3200


def _tc_body(attrs_ref, consts_ref, part_ref, out_ref):
    pv = part_ref[...]
    m_g = jnp.max(pv[0])
    s_g = jnp.sum(pv[1] * jnp.exp(pv[0] - m_g))
    inv_s = 1.0 / s_g
    cns = consts_ref[...]
    w0 = cns[0, 0]
    b0 = cns[0, 1]
    wv = cns[:, 2:22]
    bv = cns[:, 22:42]
    e0v = cns[:, 42:62]
    dev = cns[:, 62:82]
    a = attrs_ref[...]
    ty = a[:, 0:1]
    fr = a[:, 1:2]
    t = a[:, 2:3]
    lin = t * w0 + b0
    per = jnp.sin(t * wv + bv)
    wt = jnp.exp(fr - m_g) * inv_s
    wt = jnp.where(ty == 1.0, 1.0, wt)
    typef = e0v + ty * dev
    out_ref[...] = jnp.concatenate([lin, per, wt, typef], axis=1)


_tc_main = pl.pallas_call(
    _tc_body,
    grid=(E // TCB,),
    in_specs=[
        pl.BlockSpec((TCB, 3), lambda i: (i, 0)),
        pl.BlockSpec((1, 96), lambda i: (0, 0)),
        pl.BlockSpec((2, NW, 16), lambda i: (0, 0, 0)),
    ],
    out_specs=pl.BlockSpec((TCB, OUT_D), lambda i: (i, 0)),
    out_shape=jax.ShapeDtypeStruct((E, OUT_D), jnp.float32),
)


def kernel(edge_attrs, t2v_w0, t2v_b0, t2v_w, t2v_b, type_emb):
    demb = type_emb[1] - type_emb[0]
    consts = jnp.concatenate(
        [
            t2v_w0[None],
            t2v_b0[None],
            t2v_w,
            t2v_b,
            type_emb[0],
            demb,
            jnp.zeros((14,), jnp.float32),
        ]
    )
    part = _reduce_kernel(edge_attrs)
    return _tc_main(edge_attrs, consts[None, :], part)


# speedup vs baseline: 1.6950x; 1.6950x over previous
import functools

import jax
import jax.numpy as jnp
from jax import lax
from jax.experimental import pallas as pl
from jax.experimental.pallas import tpu as pltpu
from jax.experimental.pallas import tpu_sc as plsc

E = 1_600_000
K = 20
OUT_D = 42
NW = 32
EPW = E // NW
CHUNK = 80
NCHUNK = EPW // CHUNK
GROUPS = CHUNK // 16
NPAIR = NCHUNK // 2

_SMAGIC = float(1.5 * 2**23)
_INV2PI = 0.15915494309189535
_TPI_HI = 6.28125
_TPI_LO = 0.0028353071795864769
_C0 = 9.99999995e-01
_C1 = -1.66666646e-01
_C2 = 8.33331039e-03
_C3 = -1.98401553e-04
_C4 = 2.75294535e-06
_C5 = -2.46769610e-08
_C6 = 1.34514372e-10


def _sin(x):
    y = x * _INV2PI
    n = (y + _SMAGIC) - _SMAGIC
    r = x - n * _TPI_HI
    r = r - n * _TPI_LO
    r2 = r * r
    p = _C6
    p = p * r2 + _C5
    p = p * r2 + _C4
    p = p * r2 + _C3
    p = p * r2 + _C2
    p = p * r2 + _C1
    p = p * r2 + _C0
    return r * p


_MESH = plsc.VectorSubcoreMesh(core_axis_name="c", subcore_axis_name="s")


@functools.partial(
    pl.kernel,
    mesh=_MESH,
    compiler_params=pltpu.CompilerParams(needs_layout_passes=False),
    out_type=jax.ShapeDtypeStruct((2, NW, 16), jnp.float32),
    scratch_types=[
        pltpu.VMEM((2, CHUNK, 3), jnp.float32),
        pltpu.VMEM((16,), jnp.float32),
        pltpu.VMEM((16,), jnp.float32),
        pltpu.SemaphoreType.DMA,
        pltpu.SemaphoreType.DMA,
    ],
)
def _reduce_kernel(attrs_hbm, part_hbm, buf, mscr, sscr, isem0, isem1):
    wid = lax.axis_index("s") * 2 + lax.axis_index("c")
    base = wid * EPW
    iota = lax.iota(jnp.int32, 16)
    col1 = iota * 0 + 1
    isems = (isem0, isem1)

    def issue_in(cc, b):
        pltpu.async_copy(
            attrs_hbm.at[pl.ds(base + cc * CHUNK, CHUNK)], buf.at[b], isems[b]
        )

    def wait_in(cc, b):
        pltpu.make_async_copy(
            attrs_hbm.at[pl.ds(base + cc * CHUNK, CHUNK)], buf.at[b], isems[b]
        ).wait()

    def process(cc, b, carry):
        m, s = carry

        def gmax(g, cm):
            v = plsc.load_gather(buf.at[b], [iota + g * 16, col1])
            return jnp.maximum(cm, v)

        cm = lax.fori_loop(0, GROUPS, gmax, jnp.full((16,), -1e30, jnp.float32))
        mnew = jnp.maximum(m, cm)
        s = s * jnp.exp(m - mnew)

        def gsum(g, acc):
            v = plsc.load_gather(buf.at[b], [iota + g * 16, col1])
            return acc + jnp.exp(v - mnew)

        s = lax.fori_loop(0, GROUPS, gsum, s)
        return mnew, s

    issue_in(0, 0)

    def pair_body(i, carry):
        carry2 = carry
        for b in (0, 1):
            cc = 2 * i + b
            wait_in(cc, b)
            issue_in(cc + 1, 1 - b)
            carry2 = process(cc, b, carry2)
        return carry2

    m0 = jnp.full((16,), -1e30, jnp.float32)
    s0 = jnp.zeros((16,), jnp.float32)
    m, s = lax.fori_loop(0, NPAIR, pair_body, (m0, s0))
    wait_in(NCHUNK - 1, 0)
    m, s = process(NCHUNK - 1, 0, (m, s))
    mscr[...] = m
    sscr[...] = s
    pltpu.sync_copy(mscr, part_hbm.at[0, wid])
    pltpu.sync_copy(sscr, part_hbm.at[1, wid])


@functools.partial(
    pl.kernel,
    mesh=_MESH,
    compiler_params=pltpu.CompilerParams(needs_layout_passes=False),
    out_type=jax.ShapeDtypeStruct((E, OUT_D), jnp.float32),
    scratch_types=[
        pltpu.VMEM((2, CHUNK, 3), jnp.float32),
        pltpu.VMEM((2, CHUNK, OUT_D), jnp.float32),
        pltpu.VMEM((2, NW, 16), jnp.float32),
        pltpu.VMEM((96, 16), jnp.float32),
        pltpu.SemaphoreType.DMA,
        pltpu.SemaphoreType.DMA,
        pltpu.SemaphoreType.DMA,
        pltpu.SemaphoreType.DMA,
    ],
)
def _main_kernel(
    attrs_hbm, consts_hbm, part_hbm, out_hbm,
    inb, outb, pv, cv, isem0, isem1, osem0, osem1,
):
    wid = lax.axis_index("s") * 2 + lax.axis_index("c")
    base = wid * EPW
    pltpu.sync_copy(part_hbm, pv)
    pltpu.sync_copy(consts_hbm, cv)

    m = pv[0, 0]
    s = pv[1, 0]
    for i in range(1, NW):
        mi = pv[0, i]
        si = pv[1, i]
        mn = jnp.maximum(m, mi)
        s = s * jnp.exp(m - mn) + si * jnp.exp(mi - mn)
        m = mn
    m_g = jnp.broadcast_to(jnp.max(m), (16,))
    s_g = jnp.broadcast_to(jnp.sum(s * jnp.exp(m - m_g)), (16,))
    inv_s = 1.0 / s_g

    iota = lax.iota(jnp.int32, 16)
    zero = iota * 0
    cols = [zero + j for j in range(OUT_D)]
    w0 = cv[0]
    b0 = cv[1]
    wv = [cv[2 + k] for k in range(K)]
    bv = [cv[22 + k] for k in range(K)]
    e0v = [cv[42 + j] for j in range(K)]
    dev = [cv[62 + j] for j in range(K)]
    isems = (isem0, isem1)
    osems = (osem0, osem1)

    def issue_in(cc, b):
        pltpu.async_copy(
            attrs_hbm.at[pl.ds(base + cc * CHUNK, CHUNK)], inb.at[b], isems[b]
        )

    def wait_in(cc, b):
        pltpu.make_async_copy(
            attrs_hbm.at[pl.ds(base + cc * CHUNK, CHUNK)], inb.at[b], isems[b]
        ).wait()

    def issue_out(cc, b):
        pltpu.async_copy(
            outb.at[b], out_hbm.at[pl.ds(base + cc * CHUNK, CHUNK)], osems[b]
        )

    def wait_out(cc, b):
        pltpu.make_async_copy(
            outb.at[b], out_hbm.at[pl.ds(base + cc * CHUNK, CHUNK)], osems[b]
        ).wait()

    def compute(b):
        def group(g, carry2):
            rows = iota + g * 16
            ty = plsc.load_gather(inb.at[b], [rows, cols[0]])
            fr = plsc.load_gather(inb.at[b], [rows, cols[1]])
            t = plsc.load_gather(inb.at[b], [rows, cols[2]])
            plsc.store_scatter(outb.at[b], [rows, cols[0]], t * w0 + b0)
            for k in range(K):
                sv = _sin(t * wv[k] + bv[k])
                plsc.store_scatter(outb.at[b], [rows, cols[1 + k]], sv)
            e = jnp.exp(fr - m_g) * inv_s
            wcol = jnp.where(ty == 1.0, 1.0, e)
            plsc.store_scatter(outb.at[b], [rows, cols[21]], wcol)
            for j in range(K):
                col = ty * dev[j] + e0v[j]
                plsc.store_scatter(outb.at[b], [rows, cols[22 + j]], col)
            return carry2

        lax.fori_loop(0, GROUPS, group, 0)

    issue_in(0, 0)

    def pair_body(i, carry):
        for b in (0, 1):
            cc = 2 * i + b
            wait_in(cc, b)
            issue_in(cc + 1, 1 - b)

            @pl.when(i >= 1)
            def _():
                wait_out(cc - 2, b)

            compute(b)
            issue_out(cc, b)
        return carry

    lax.fori_loop(0, NPAIR, pair_body, 0)
    cc_last = NCHUNK - 1
    wait_in(cc_last, 0)
    wait_out(cc_last - 2, 0)
    compute(0)
    issue_out(cc_last, 0)
    wait_out(cc_last - 1, 1)
    wait_out(cc_last, 0)


def kernel(edge_attrs, t2v_w0, t2v_b0, t2v_w, t2v_b, type_emb):
    demb = type_emb[1] - type_emb[0]
    consts = jnp.concatenate(
        [
            t2v_w0[None],
            t2v_b0[None],
            t2v_w,
            t2v_b,
            type_emb[0],
            demb,
            jnp.zeros((14,), jnp.float32),
        ]
    )
    consts = jnp.tile(consts[:, None], (1, 16))
    part = _reduce_kernel(edge_attrs)
    return _main_kernel(edge_attrs, consts, part)


# speedup vs baseline: 1.8973x; 1.1193x over previous
import functools

import jax
import jax.numpy as jnp
from jax import lax
from jax.experimental import pallas as pl
from jax.experimental.pallas import tpu as pltpu
from jax.experimental.pallas import tpu_sc as plsc

E = 1_600_000
K = 20
OUT_D = 42
NW = 32
EPW = E // NW

CHUNK_R = 400
NCHUNK_R = EPW // CHUNK_R
GROUPS_R = CHUNK_R // 16
NPAIR_R = NCHUNK_R // 2

CHUNK = 80
NCHUNK = EPW // CHUNK
GROUPS = CHUNK // 16
NBUF = 4
NQUAD = NCHUNK // NBUF

_SMAGIC = float(1.5 * 2**23)
_INV2PI = 0.15915494309189535
_TPI_HI = 6.28125
_TPI_LO = 0.0028353071795864769
_C0 = 9.99999995e-01
_C1 = -1.66666646e-01
_C2 = 8.33331039e-03
_C3 = -1.98401553e-04
_C4 = 2.75294535e-06
_C5 = -2.46769610e-08
_C6 = 1.34514372e-10


def _sin(x):
    y = x * _INV2PI
    n = (y + _SMAGIC) - _SMAGIC
    r = x - n * _TPI_HI
    r = r - n * _TPI_LO
    r2 = r * r
    p = _C6
    p = p * r2 + _C5
    p = p * r2 + _C4
    p = p * r2 + _C3
    p = p * r2 + _C2
    p = p * r2 + _C1
    p = p * r2 + _C0
    return r * p


_MESH = plsc.VectorSubcoreMesh(core_axis_name="c", subcore_axis_name="s")


@functools.partial(
    pl.kernel,
    mesh=_MESH,
    compiler_params=pltpu.CompilerParams(needs_layout_passes=False),
    out_type=jax.ShapeDtypeStruct((2, NW, 16), jnp.float32),
    scratch_types=[
        pltpu.VMEM((2, CHUNK_R, 3), jnp.float32),
        pltpu.VMEM((16,), jnp.float32),
        pltpu.VMEM((16,), jnp.float32),
        pltpu.SemaphoreType.DMA,
        pltpu.SemaphoreType.DMA,
    ],
)
def _reduce_kernel(attrs_hbm, part_hbm, buf, mscr, sscr, isem0, isem1):
    wid = lax.axis_index("s") * 2 + lax.axis_index("c")
    base = wid * EPW
    iota = lax.iota(jnp.int32, 16)
    col1 = iota * 0 + 1
    isems = (isem0, isem1)

    def issue_in(cc, b):
        pltpu.async_copy(
            attrs_hbm.at[pl.ds(base + cc * CHUNK_R, CHUNK_R)], buf.at[b], isems[b]
        )

    def wait_in(cc, b):
        pltpu.make_async_copy(
            attrs_hbm.at[pl.ds(base + cc * CHUNK_R, CHUNK_R)], buf.at[b], isems[b]
        ).wait()

    def process(cc, b, carry):
        m, s = carry

        def gmax(g, cm):
            v = plsc.load_gather(buf.at[b], [iota + g * 16, col1])
            return jnp.maximum(cm, v)

        cm = lax.fori_loop(0, GROUPS_R, gmax, jnp.full((16,), -1e30, jnp.float32))
        mnew = jnp.maximum(m, cm)
        s = s * jnp.exp(m - mnew)

        def gsum(g, acc):
            v = plsc.load_gather(buf.at[b], [iota + g * 16, col1])
            return acc + jnp.exp(v - mnew)

        s = lax.fori_loop(0, GROUPS_R, gsum, s)
        return mnew, s

    issue_in(0, 0)

    def pair_body(i, carry):
        carry2 = carry
        for b in (0, 1):
            cc = 2 * i + b
            wait_in(cc, b)
            issue_in(cc + 1, 1 - b)
            carry2 = process(cc, b, carry2)
        return carry2

    m0 = jnp.full((16,), -1e30, jnp.float32)
    s0 = jnp.zeros((16,), jnp.float32)
    m, s = lax.fori_loop(0, NPAIR_R, pair_body, (m0, s0))
    wait_in(NCHUNK_R - 1, 0)
    m, s = process(NCHUNK_R - 1, 0, (m, s))
    mscr[...] = m
    sscr[...] = s
    pltpu.sync_copy(mscr, part_hbm.at[0, wid])
    pltpu.sync_copy(sscr, part_hbm.at[1, wid])


@functools.partial(
    pl.kernel,
    mesh=_MESH,
    compiler_params=pltpu.CompilerParams(needs_layout_passes=False),
    out_type=jax.ShapeDtypeStruct((E, OUT_D), jnp.float32),
    scratch_types=[
        pltpu.VMEM((NBUF, CHUNK, 3), jnp.float32),
        pltpu.VMEM((NBUF, CHUNK, OUT_D), jnp.float32),
        pltpu.VMEM((2, NW, 16), jnp.float32),
        pltpu.VMEM((96, 16), jnp.float32),
        [pltpu.SemaphoreType.DMA] * NBUF,
        [pltpu.SemaphoreType.DMA] * NBUF,
    ],
)
def _main_kernel(
    attrs_hbm, consts_hbm, part_hbm, out_hbm, inb, outb, pv, cv, isems, osems
):
    wid = lax.axis_index("s") * 2 + lax.axis_index("c")
    base = wid * EPW
    pltpu.sync_copy(part_hbm, pv)
    pltpu.sync_copy(consts_hbm, cv)

    m = pv[0, 0]
    s = pv[1, 0]
    for i in range(1, NW):
        mi = pv[0, i]
        si = pv[1, i]
        mn = jnp.maximum(m, mi)
        s = s * jnp.exp(m - mn) + si * jnp.exp(mi - mn)
        m = mn
    m_g = jnp.broadcast_to(jnp.max(m), (16,))
    s_g = jnp.broadcast_to(jnp.sum(s * jnp.exp(m - m_g)), (16,))
    inv_s = 1.0 / s_g

    iota = lax.iota(jnp.int32, 16)
    zero = iota * 0
    cols = [zero + j for j in range(OUT_D)]
    w0 = cv[0]
    b0 = cv[1]
    wv = [cv[2 + k] for k in range(K)]
    bv = [cv[22 + k] for k in range(K)]
    e0v = [cv[42 + j] for j in range(K)]
    dev = [cv[62 + j] for j in range(K)]

    def issue_in(cc, b):
        pltpu.async_copy(
            attrs_hbm.at[pl.ds(base + cc * CHUNK, CHUNK)], inb.at[b], isems[b]
        )

    def wait_in(cc, b):
        pltpu.make_async_copy(
            attrs_hbm.at[pl.ds(base + cc * CHUNK, CHUNK)], inb.at[b], isems[b]
        ).wait()

    def issue_out(cc, b):
        pltpu.async_copy(
            outb.at[b], out_hbm.at[pl.ds(base + cc * CHUNK, CHUNK)], osems[b]
        )

    def wait_out(cc, b):
        pltpu.make_async_copy(
            outb.at[b], out_hbm.at[pl.ds(base + cc * CHUNK, CHUNK)], osems[b]
        ).wait()

    def compute(b):
        def group(g, carry2):
            rows = iota + g * 16
            ty = plsc.load_gather(inb.at[b], [rows, cols[0]])
            fr = plsc.load_gather(inb.at[b], [rows, cols[1]])
            t = plsc.load_gather(inb.at[b], [rows, cols[2]])
            plsc.store_scatter(outb.at[b], [rows, cols[0]], t * w0 + b0)
            for k in range(K):
                sv = _sin(t * wv[k] + bv[k])
                plsc.store_scatter(outb.at[b], [rows, cols[1 + k]], sv)
            e = jnp.exp(fr - m_g) * inv_s
            wcol = jnp.where(ty == 1.0, 1.0, e)
            plsc.store_scatter(outb.at[b], [rows, cols[21]], wcol)
            for j in range(K):
                col = ty * dev[j] + e0v[j]
                plsc.store_scatter(outb.at[b], [rows, cols[22 + j]], col)
            return carry2

        lax.fori_loop(0, GROUPS, group, 0)

    for b in range(NBUF - 1):
        issue_in(b, b)

    def quad_body(i, carry):
        for b in range(NBUF):
            cc = NBUF * i + b
            wait_in(cc, b)

            @pl.when(cc + NBUF - 1 <= NCHUNK - 1)
            def _():
                issue_in(cc + NBUF - 1, (b + NBUF - 1) % NBUF)

            @pl.when(i >= 1)
            def _():
                wait_out(cc - NBUF, b)

            compute(b)
            issue_out(cc, b)
        return carry

    lax.fori_loop(0, NQUAD, quad_body, 0)
    cc_last = NCHUNK - 1
    wait_in(cc_last, 0)
    wait_out(cc_last - NBUF, 0)
    compute(0)
    issue_out(cc_last, 0)
    for b in range(1, NBUF):
        wait_out(cc_last - NBUF + b, b)
    wait_out(cc_last, 0)


def kernel(edge_attrs, t2v_w0, t2v_b0, t2v_w, t2v_b, type_emb):
    demb = type_emb[1] - type_emb[0]
    consts = jnp.concatenate(
        [
            t2v_w0[None],
            t2v_b0[None],
            t2v_w,
            t2v_b,
            type_emb[0],
            demb,
            jnp.zeros((14,), jnp.float32),
        ]
    )
    consts = jnp.tile(consts[:, None], (1, 16))
    part = _reduce_kernel(edge_attrs)
    return _main_kernel(edge_attrs, consts, part)


# speedup vs baseline: 1.9007x; 1.0018x over previous
import functools

import jax
import jax.numpy as jnp
from jax import lax
from jax.experimental import pallas as pl
from jax.experimental.pallas import tpu as pltpu
from jax.experimental.pallas import tpu_sc as plsc

E = 1_600_000
K = 20
OUT_D = 42
NW = 32
EPW = E // NW

CHUNK_R = 400
NCHUNK_R = EPW // CHUNK_R
GROUPS_R = CHUNK_R // 16
NPAIR_R = NCHUNK_R // 2

CHUNK = 80
NCHUNK = EPW // CHUNK
GROUPS = CHUNK // 16
NBUF = 5
NQUAD = NCHUNK // NBUF

_SMAGIC = float(1.5 * 2**23)
_INV2PI = 0.15915494309189535
_TPI_HI = 6.28125
_TPI_LO = 0.0028353071795864769
_C0 = 9.99999995e-01
_C1 = -1.66666646e-01
_C2 = 8.33331039e-03
_C3 = -1.98401553e-04
_C4 = 2.75294535e-06
_C5 = -2.46769610e-08
_C6 = 1.34514372e-10


def _sin(x):
    y = x * _INV2PI
    n = (y + _SMAGIC) - _SMAGIC
    r = x - n * _TPI_HI
    r = r - n * _TPI_LO
    r2 = r * r
    p = _C6
    p = p * r2 + _C5
    p = p * r2 + _C4
    p = p * r2 + _C3
    p = p * r2 + _C2
    p = p * r2 + _C1
    p = p * r2 + _C0
    return r * p


_MESH = plsc.VectorSubcoreMesh(core_axis_name="c", subcore_axis_name="s")


@functools.partial(
    pl.kernel,
    mesh=_MESH,
    compiler_params=pltpu.CompilerParams(needs_layout_passes=False),
    out_type=jax.ShapeDtypeStruct((2, NW, 16), jnp.float32),
    scratch_types=[
        pltpu.VMEM((2, CHUNK_R, 3), jnp.float32),
        pltpu.VMEM((16,), jnp.float32),
        pltpu.VMEM((16,), jnp.float32),
        pltpu.SemaphoreType.DMA,
        pltpu.SemaphoreType.DMA,
    ],
)
def _reduce_kernel(attrs_hbm, part_hbm, buf, mscr, sscr, isem0, isem1):
    wid = lax.axis_index("s") * 2 + lax.axis_index("c")
    base = wid * EPW
    iota = lax.iota(jnp.int32, 16)
    col1 = iota * 0 + 1
    isems = (isem0, isem1)

    def issue_in(cc, b):
        pltpu.async_copy(
            attrs_hbm.at[pl.ds(base + cc * CHUNK_R, CHUNK_R)], buf.at[b], isems[b]
        )

    def wait_in(cc, b):
        pltpu.make_async_copy(
            attrs_hbm.at[pl.ds(base + cc * CHUNK_R, CHUNK_R)], buf.at[b], isems[b]
        ).wait()

    def process(cc, b, carry):
        m, s = carry

        def gmax(g, cm):
            v = plsc.load_gather(buf.at[b], [iota + g * 16, col1])
            return jnp.maximum(cm, v)

        cm = lax.fori_loop(0, GROUPS_R, gmax, jnp.full((16,), -1e30, jnp.float32))
        mnew = jnp.maximum(m, cm)
        s = s * jnp.exp(m - mnew)

        def gsum(g, acc):
            v = plsc.load_gather(buf.at[b], [iota + g * 16, col1])
            return acc + jnp.exp(v - mnew)

        s = lax.fori_loop(0, GROUPS_R, gsum, s)
        return mnew, s

    issue_in(0, 0)

    def pair_body(i, carry):
        carry2 = carry
        for b in (0, 1):
            cc = 2 * i + b
            wait_in(cc, b)
            issue_in(cc + 1, 1 - b)
            carry2 = process(cc, b, carry2)
        return carry2

    m0 = jnp.full((16,), -1e30, jnp.float32)
    s0 = jnp.zeros((16,), jnp.float32)
    m, s = lax.fori_loop(0, NPAIR_R, pair_body, (m0, s0))
    wait_in(NCHUNK_R - 1, 0)
    m, s = process(NCHUNK_R - 1, 0, (m, s))
    mscr[...] = m
    sscr[...] = s
    pltpu.sync_copy(mscr, part_hbm.at[0, wid])
    pltpu.sync_copy(sscr, part_hbm.at[1, wid])


@functools.partial(
    pl.kernel,
    mesh=_MESH,
    compiler_params=pltpu.CompilerParams(needs_layout_passes=False),
    out_type=jax.ShapeDtypeStruct((E, OUT_D), jnp.float32),
    scratch_types=[
        pltpu.VMEM((NBUF, CHUNK, 3), jnp.float32),
        pltpu.VMEM((NBUF, CHUNK, OUT_D), jnp.float32),
        pltpu.VMEM((2, NW, 16), jnp.float32),
        pltpu.VMEM((96, 16), jnp.float32),
        [pltpu.SemaphoreType.DMA] * NBUF,
        [pltpu.SemaphoreType.DMA] * NBUF,
    ],
)
def _main_kernel(
    attrs_hbm, consts_hbm, part_hbm, out_hbm, inb, outb, pv, cv, isems, osems
):
    wid = lax.axis_index("s") * 2 + lax.axis_index("c")
    base = wid * EPW
    pltpu.sync_copy(part_hbm, pv)
    pltpu.sync_copy(consts_hbm, cv)

    m = pv[0, 0]
    s = pv[1, 0]
    for i in range(1, NW):
        mi = pv[0, i]
        si = pv[1, i]
        mn = jnp.maximum(m, mi)
        s = s * jnp.exp(m - mn) + si * jnp.exp(mi - mn)
        m = mn
    m_g = jnp.broadcast_to(jnp.max(m), (16,))
    s_g = jnp.broadcast_to(jnp.sum(s * jnp.exp(m - m_g)), (16,))
    inv_s = 1.0 / s_g

    iota = lax.iota(jnp.int32, 16)
    zero = iota * 0
    cols = [zero + j for j in range(OUT_D)]
    w0 = cv[0]
    b0 = cv[1]
    wv = [cv[2 + k] for k in range(K)]
    bv = [cv[22 + k] for k in range(K)]
    e0v = [cv[42 + j] for j in range(K)]
    dev = [cv[62 + j] for j in range(K)]

    def issue_in(cc, b):
        pltpu.async_copy(
            attrs_hbm.at[pl.ds(base + cc * CHUNK, CHUNK)], inb.at[b], isems[b]
        )

    def wait_in(cc, b):
        pltpu.make_async_copy(
            attrs_hbm.at[pl.ds(base + cc * CHUNK, CHUNK)], inb.at[b], isems[b]
        ).wait()

    def issue_out(cc, b):
        pltpu.async_copy(
            outb.at[b], out_hbm.at[pl.ds(base + cc * CHUNK, CHUNK)], osems[b]
        )

    def wait_out(cc, b):
        pltpu.make_async_copy(
            outb.at[b], out_hbm.at[pl.ds(base + cc * CHUNK, CHUNK)], osems[b]
        ).wait()

    def compute(b):
        def group(g, carry2):
            rows = iota + g * 16
            ty = plsc.load_gather(inb.at[b], [rows, cols[0]])
            fr = plsc.load_gather(inb.at[b], [rows, cols[1]])
            t = plsc.load_gather(inb.at[b], [rows, cols[2]])
            plsc.store_scatter(outb.at[b], [rows, cols[0]], t * w0 + b0)
            for k in range(K):
                sv = _sin(t * wv[k] + bv[k])
                plsc.store_scatter(outb.at[b], [rows, cols[1 + k]], sv)
            e = jnp.exp(fr - m_g) * inv_s
            wcol = jnp.where(ty == 1.0, 1.0, e)
            plsc.store_scatter(outb.at[b], [rows, cols[21]], wcol)
            for j in range(K):
                col = ty * dev[j] + e0v[j]
                plsc.store_scatter(outb.at[b], [rows, cols[22 + j]], col)
            return carry2

        lax.fori_loop(0, GROUPS, group, 0)

    for b in range(NBUF - 1):
        issue_in(b, b)

    def quad_body(i, carry):
        for b in range(NBUF):
            cc = NBUF * i + b
            wait_in(cc, b)

            @pl.when(cc + NBUF - 1 <= NCHUNK - 1)
            def _():
                issue_in(cc + NBUF - 1, (b + NBUF - 1) % NBUF)

            @pl.when(i >= 1)
            def _():
                wait_out(cc - NBUF, b)

            compute(b)
            issue_out(cc, b)
        return carry

    lax.fori_loop(0, NQUAD, quad_body, 0)
    for b in range(NBUF):
        wait_out(NCHUNK - NBUF + b, b)


def kernel(edge_attrs, t2v_w0, t2v_b0, t2v_w, t2v_b, type_emb):
    demb = type_emb[1] - type_emb[0]
    consts = jnp.concatenate(
        [
            t2v_w0[None],
            t2v_b0[None],
            t2v_w,
            t2v_b,
            type_emb[0],
            demb,
            jnp.zeros((14,), jnp.float32),
        ]
    )
    consts = jnp.tile(consts[:, None], (1, 16))
    part = _reduce_kernel(edge_attrs)
    return _main_kernel(edge_attrs, consts, part)
